# baseline (speedup 1.0000x reference)
import jax
import jax.numpy as jnp
from jax.experimental import pallas as pl
from jax.experimental.pallas import tpu as pltpu

BATCH = 16384
EMBED_DIM = 128
N_TOTAL = float(BATCH * EMBED_DIM)
SEG_ROWS = 1024
NSEG = BATCH // SEG_ROWS
NBUF = 4


def _nll_kernel(cw_hbm, xw_hbm, out_ref, abuf, bbuf, acc_ref, sems):
    def issue(seg, slot):
        rows = pl.ds(seg * SEG_ROWS, SEG_ROWS)
        da = pltpu.async_copy(cw_hbm.at[rows, :], abuf.at[slot],
                              sems.at[slot, 0])
        db = pltpu.async_copy(xw_hbm.at[rows, :], bbuf.at[slot],
                              sems.at[slot, 1])
        return da, db

    acc_ref[0] = 0.0
    acc_ref[1] = 0.0

    descs = [issue(s, s) for s in range(NBUF)]
    for s in range(NSEG):
        slot = s % NBUF
        da, db = descs[slot]
        da.wait()
        db.wait()
        mul = abuf[slot] * bbuf[slot]
        if s + NBUF < NSEG:
            descs[slot] = issue(s + NBUF, slot)
        acc_ref[0] += jnp.sum(jnp.exp(mul))
        acc_ref[1] += jnp.sum(mul)

    out_ref[0] = N_TOTAL * jnp.log(acc_ref[0]) - acc_ref[1]


@jax.jit
def kernel(center_word, context_word, center_emb, context_emb):
    del center_emb, context_emb
    out = pl.pallas_call(
        _nll_kernel,
        in_specs=[
            pl.BlockSpec(memory_space=pltpu.HBM),
            pl.BlockSpec(memory_space=pltpu.HBM),
        ],
        out_specs=pl.BlockSpec(memory_space=pltpu.SMEM),
        out_shape=jax.ShapeDtypeStruct((1,), jnp.float32),
        scratch_shapes=[
            pltpu.VMEM((NBUF, SEG_ROWS, EMBED_DIM), jnp.float32),
            pltpu.VMEM((NBUF, SEG_ROWS, EMBED_DIM), jnp.float32),
            pltpu.SMEM((2,), jnp.float32),
            pltpu.SemaphoreType.DMA((NBUF, 2)),
        ],
    )(center_word, context_word)
    return out[0]


# speedup vs baseline: 1.0330x; 1.0330x over previous
import jax
import jax.numpy as jnp
from jax.experimental import pallas as pl
from jax.experimental.pallas import tpu as pltpu

BATCH = 16384
EMBED_DIM = 128
N_TOTAL = float(BATCH * EMBED_DIM)
BLOCK_ROWS = 8192
GRID = BATCH // BLOCK_ROWS


def _nll_kernel(cw_ref, xw_ref, out_ref, acc_ref):
    i = pl.program_id(0)

    @pl.when(i == 0)
    def _init():
        acc_ref[0] = 0.0
        acc_ref[1] = 0.0

    mul = cw_ref[...] * xw_ref[...]
    acc_ref[0] += jnp.sum(jnp.exp(mul))
    acc_ref[1] += jnp.sum(mul)

    @pl.when(i == GRID - 1)
    def _fini():
        out_ref[0] = N_TOTAL * jnp.log(acc_ref[0]) - acc_ref[1]


@jax.jit
def kernel(center_word, context_word, center_emb, context_emb):
    del center_emb, context_emb
    out = pl.pallas_call(
        _nll_kernel,
        grid=(GRID,),
        in_specs=[
            pl.BlockSpec((BLOCK_ROWS, EMBED_DIM), lambda i: (i, 0)),
            pl.BlockSpec((BLOCK_ROWS, EMBED_DIM), lambda i: (i, 0)),
        ],
        out_specs=pl.BlockSpec(memory_space=pltpu.SMEM),
        out_shape=jax.ShapeDtypeStruct((1,), jnp.float32),
        scratch_shapes=[pltpu.SMEM((2,), jnp.float32)],
    )(center_word, context_word)
    return out[0]
